# baseline (speedup 1.0000x reference)
import jax
import jax.numpy as jnp
import numpy as np
from jax import lax
from jax.experimental import pallas as pl
from jax.experimental.pallas import tpu as pltpu
from jax.experimental.pallas import tpu_sc as plsc

N_NODES = 10000
IN_DIM = 128
HIDDEN = 2000
HPAD = 2048
L_DIM = 64
L_PAD = 128
N_EDGES = 320000

NC = 2
NS = 16
NW = NC * NS
CH = 128
NCHUNK = 80
EPT = NCHUNK * CH
E_PAD = NW * EPT
NPAD = 10240
GRP = 16
RB = 512
GRID = NPAD // RB
RPS = NPAD // NS
DEG_W = 16

_MESH = plsc.VectorSubcoreMesh(
    core_axis_name="c", subcore_axis_name="s", num_cores=NC, num_subcores=NS)


def _sc_degree_body(dst_hbm, out_hbm, idxd_v, ones_v, stage_v, col_v,
                    acc_sh, sem):
    c = lax.axis_index("c")
    s = lax.axis_index("s")
    tid = c * NS + s
    zeros16 = jnp.zeros((16,), jnp.float32)
    ones16 = jnp.ones((16,), jnp.float32)
    iota16 = jnp.arange(16, dtype=jnp.int32)
    zeros16i = jnp.zeros((16,), jnp.int32)

    pltpu.sync_copy(dst_hbm.at[tid], idxd_v)

    @pl.loop(0, CH)
    def _zrow(i):
        ones_v[i, pl.ds(0, DEG_W)] = zeros16

    for k in range(RPS // CH):
        pltpu.sync_copy(ones_v, acc_sh.at[pl.ds(s * RPS + k * CH, CH)])

    @pl.loop(0, CH)
    def _onerow(i):
        ones_v[i, pl.ds(0, DEG_W)] = ones16

    plsc.subcore_barrier()

    @pl.loop(0, NCHUNK // 8)
    def _grp(g):
        descs = [
            pltpu.async_copy(ones_v, acc_sh.at[idxd_v.at[g * 8 + b]], sem,
                             add=True)
            for b in range(8)
        ]
        for d in descs:
            d.wait()

    plsc.subcore_barrier()
    for k in range(RPS // CH):
        pltpu.sync_copy(acc_sh.at[pl.ds(s * RPS + k * CH, CH)], stage_v)

        @pl.loop(0, CH // 16)
        def _ext(g):
            lane = jnp.arange(16, dtype=jnp.int32)
            col = jnp.zeros((16,), jnp.float32)
            for r in range(16):
                v = stage_v[g * 16 + r, pl.ds(0, 16)]
                col = jnp.where(lane == r, jnp.full((16,), v[0]), col)
            col_v[pl.ds(k * CH + g * 16, 16)] = col

    pltpu.sync_copy(col_v, out_hbm.at[c, pl.ds(s * RPS, RPS)])


_sc_degree = pl.kernel(
    _sc_degree_body,
    out_type=jax.ShapeDtypeStruct((NC, NPAD), jnp.float32),
    mesh=_MESH,
    scratch_types=[
        pltpu.VMEM((NCHUNK, CH), jnp.int32),
        pltpu.VMEM((CH, DEG_W), jnp.float32),
        pltpu.VMEM((CH, DEG_W), jnp.float32),
        pltpu.VMEM((RPS,), jnp.float32),
        pltpu.VMEM_SHARED((NPAD, DEG_W), jnp.float32),
        pltpu.SemaphoreType.DMA,
    ],
)


def _make_sc_scatter(d):
    def body(y_hbm, src_hbm, dst_hbm, out_hbm,
             idxs_v, idxd_v, r0, r1, acc_sh, s0, s1):
        rows = [r0, r1]
        sems = [s0, s1]
        c = lax.axis_index("c")
        s = lax.axis_index("s")
        tid = c * NS + s
        zeros16 = jnp.zeros((16,), jnp.float32)

        @pl.loop(0, CH)
        def _zrow(i):
            for j in range(d // 16):
                r0[i, pl.ds(j * 16, 16)] = zeros16

        for k in range(RPS // CH):
            pltpu.sync_copy(r0, acc_sh.at[pl.ds(s * RPS + k * CH, CH)])
        plsc.subcore_barrier()

        @pl.loop(0, NCHUNK // GRP)
        def _grp(g):
            pltpu.sync_copy(src_hbm.at[tid, pl.ds(g * GRP, GRP)], idxs_v)
            pltpu.sync_copy(dst_hbm.at[tid, pl.ds(g * GRP, GRP)], idxd_v)
            descs = [
                pltpu.async_copy(y_hbm.at[idxs_v.at[b]], rows[b], sems[b])
                for b in range(2)
            ]
            for t in range(GRP):
                b = t % 2
                descs[b].wait()
                pltpu.sync_copy(rows[b], acc_sh.at[idxd_v.at[t]], add=True)
                if t + 2 < GRP:
                    descs[b] = pltpu.async_copy(y_hbm.at[idxs_v.at[t + 2]],
                                                rows[b], sems[b])

        plsc.subcore_barrier()
        pltpu.sync_copy(acc_sh.at[pl.ds(s * RPS, RPS)],
                        out_hbm.at[c, pl.ds(s * RPS, RPS)])

    return pl.kernel(
        body,
        out_type=jax.ShapeDtypeStruct((NC, NPAD, d), jnp.float32),
        mesh=_MESH,
        scratch_types=[
            pltpu.VMEM((GRP, CH), jnp.int32),
            pltpu.VMEM((GRP, CH), jnp.int32),
            pltpu.VMEM((CH, d), jnp.float32),
            pltpu.VMEM((CH, d), jnp.float32),
            pltpu.VMEM_SHARED((NPAD, d), jnp.float32),
            pltpu.SemaphoreType.DMA,
            pltpu.SemaphoreType.DMA,
        ],
    )


_sc_scatter_128 = _make_sc_scatter(IN_DIM)


def _prep_body(deg_ref, x_ref, y_ref, dis_ref):
    deg = jnp.sum(deg_ref[...], axis=0) + 1.0
    dis = lax.rsqrt(deg)
    dis_ref[...] = dis
    y_ref[...] = dis * x_ref[...]


def _tc_prep(deg_parts, xp):
    return pl.pallas_call(
        _prep_body,
        grid=(GRID,),
        in_specs=[
            pl.BlockSpec((NC, RB, 1), lambda r: (0, r, 0)),
            pl.BlockSpec((RB, IN_DIM), lambda r: (r, 0)),
        ],
        out_specs=[
            pl.BlockSpec((RB, IN_DIM), lambda r: (r, 0)),
            pl.BlockSpec((RB, 1), lambda r: (r, 0)),
        ],
        out_shape=[
            jax.ShapeDtypeStruct((NPAD, IN_DIM), jnp.float32),
            jax.ShapeDtypeStruct((NPAD, 1), jnp.float32),
        ],
    )(deg_parts, xp)


def _fused_body(acc_ref, y_ref, dis_ref, w1_ref, b1_ref, w2_ref, ym_ref):
    dis = dis_ref[...]
    px = dis * (jnp.sum(acc_ref[...], axis=0) + y_ref[...])
    h = jnp.dot(px, w1_ref[...], preferred_element_type=jnp.float32)
    h = jnp.maximum(h + b1_ref[...], 0.0)
    m = jnp.dot(h, w2_ref[...], preferred_element_type=jnp.float32)
    ym_ref[...] = dis * m


def _tc_fused(acc, y, dis, w1p, b1p, w2p):
    return pl.pallas_call(
        _fused_body,
        grid=(GRID,),
        in_specs=[
            pl.BlockSpec((NC, RB, IN_DIM), lambda r: (0, r, 0)),
            pl.BlockSpec((RB, IN_DIM), lambda r: (r, 0)),
            pl.BlockSpec((RB, 1), lambda r: (r, 0)),
            pl.BlockSpec((IN_DIM, HPAD), lambda r: (0, 0)),
            pl.BlockSpec((1, HPAD), lambda r: (0, 0)),
            pl.BlockSpec((HPAD, L_PAD), lambda r: (0, 0)),
        ],
        out_specs=pl.BlockSpec((RB, L_PAD), lambda r: (r, 0)),
        out_shape=jax.ShapeDtypeStruct((NPAD, L_PAD), jnp.float32),
    )(acc, y, dis, w1p, b1p, w2p)


def _final_body(acc_ref, ym_ref, dis_ref, b2_ref, out_ref):
    acc = dis_ref[...] * (jnp.sum(acc_ref[...], axis=0) + ym_ref[...])
    out_ref[...] = jax.nn.sigmoid(acc + b2_ref[...])[:, :L_DIM]


def _tc_final(acc, ym, dis, b2r):
    return pl.pallas_call(
        _final_body,
        grid=(GRID,),
        in_specs=[
            pl.BlockSpec((NC, RB, L_PAD), lambda r: (0, r, 0)),
            pl.BlockSpec((RB, L_PAD), lambda r: (r, 0)),
            pl.BlockSpec((RB, 1), lambda r: (r, 0)),
            pl.BlockSpec((1, L_PAD), lambda r: (0, 0)),
        ],
        out_specs=pl.BlockSpec((RB, L_DIM), lambda r: (r, 0)),
        out_shape=jax.ShapeDtypeStruct((N_NODES, L_DIM), jnp.float32),
    )(acc, ym, dis, b2r)


@jax.jit
def kernel(x, edge_index, W1, b1, W2, b2):
    src = edge_index[0].astype(jnp.int32)
    dst = edge_index[1].astype(jnp.int32)
    pad = N_NODES + (jnp.arange(E_PAD - N_EDGES, dtype=jnp.int32)
                     % (NPAD - N_NODES))
    srcp = jnp.concatenate([src, pad]).reshape(NW, NCHUNK, CH)
    dstp = jnp.concatenate([dst, pad]).reshape(NW, NCHUNK, CH)

    w1p = jnp.pad(W1.astype(jnp.float32), ((0, 0), (0, HPAD - HIDDEN)))
    b1p = jnp.pad(b1.astype(jnp.float32), (0, HPAD - HIDDEN)).reshape(1, HPAD)
    w2p = jnp.pad(W2.astype(jnp.float32),
                  ((0, HPAD - HIDDEN), (0, L_PAD - L_DIM)))
    b2r = jnp.pad(b2.astype(jnp.float32), (0, L_PAD - L_DIM)).reshape(1, L_PAD)

    xp = jnp.pad(x.astype(jnp.float32), ((0, NPAD - N_NODES), (0, 0)))
    deg_parts = _sc_degree(dstp).reshape(NC, NPAD, 1)
    y, dis = _tc_prep(deg_parts, xp)
    acc1 = _sc_scatter_128(y, srcp, dstp)
    ym = _tc_fused(acc1, y, dis, w1p, b1p, w2p)
    acc2 = _sc_scatter_128(ym, srcp, dstp)
    return _tc_final(acc2, ym, dis, b2r)


# speedup vs baseline: 1.0462x; 1.0462x over previous
import jax
import jax.numpy as jnp
import numpy as np
from jax import lax
from jax.experimental import pallas as pl
from jax.experimental.pallas import tpu as pltpu
from jax.experimental.pallas import tpu_sc as plsc

N_NODES = 10000
IN_DIM = 128
HIDDEN = 2000
HPAD = 2048
L_DIM = 64
L_PAD = 128
N_EDGES = 320000

NC = 2
NS = 16
NW = NC * NS
CH = 128
NCHUNK = 80
EPT = NCHUNK * CH
E_PAD = NW * EPT
NPAD = 10240
GRP = 40
RB = 512
GRID = NPAD // RB
RPS = NPAD // NS
DEG_W = 16

_MESH = plsc.VectorSubcoreMesh(
    core_axis_name="c", subcore_axis_name="s", num_cores=NC, num_subcores=NS)


def _sc_degree_body(dst_hbm, out_hbm, idxd_v, ones_v, stage_v, col_v,
                    acc_sh, sem):
    c = lax.axis_index("c")
    s = lax.axis_index("s")
    tid = c * NS + s
    zeros16 = jnp.zeros((16,), jnp.float32)
    ones16 = jnp.ones((16,), jnp.float32)
    iota16 = jnp.arange(16, dtype=jnp.int32)
    zeros16i = jnp.zeros((16,), jnp.int32)

    pltpu.sync_copy(dst_hbm.at[tid], idxd_v)

    @pl.loop(0, CH)
    def _zrow(i):
        ones_v[i, pl.ds(0, DEG_W)] = zeros16

    for k in range(RPS // CH):
        pltpu.sync_copy(ones_v, acc_sh.at[pl.ds(s * RPS + k * CH, CH)])

    @pl.loop(0, CH)
    def _onerow(i):
        ones_v[i, pl.ds(0, DEG_W)] = ones16

    plsc.subcore_barrier()

    @pl.loop(0, NCHUNK // 8)
    def _grp(g):
        descs = [
            pltpu.async_copy(ones_v, acc_sh.at[idxd_v.at[g * 8 + b]], sem,
                             add=True)
            for b in range(8)
        ]
        for d in descs:
            d.wait()

    plsc.subcore_barrier()
    for k in range(RPS // CH):
        pltpu.sync_copy(acc_sh.at[pl.ds(s * RPS + k * CH, CH)], stage_v)

        @pl.loop(0, CH // 16)
        def _ext(g):
            lane = jnp.arange(16, dtype=jnp.int32)
            col = jnp.zeros((16,), jnp.float32)
            for r in range(16):
                v = stage_v[g * 16 + r, pl.ds(0, 16)]
                col = jnp.where(lane == r, jnp.full((16,), v[0]), col)
            col_v[pl.ds(k * CH + g * 16, 16)] = col

    pltpu.sync_copy(col_v, out_hbm.at[c, pl.ds(s * RPS, RPS)])


_sc_degree = pl.kernel(
    _sc_degree_body,
    out_type=jax.ShapeDtypeStruct((NC, NPAD), jnp.float32),
    mesh=_MESH,
    scratch_types=[
        pltpu.VMEM((NCHUNK, CH), jnp.int32),
        pltpu.VMEM((CH, DEG_W), jnp.float32),
        pltpu.VMEM((CH, DEG_W), jnp.float32),
        pltpu.VMEM((RPS,), jnp.float32),
        pltpu.VMEM_SHARED((NPAD, DEG_W), jnp.float32),
        pltpu.SemaphoreType.DMA,
    ],
)


def _make_sc_scatter(d):
    def body(y_hbm, src_hbm, dst_hbm, out_hbm,
             idxs_v, idxd_v, r0, r1, acc_sh, s0, s1):
        rows = [r0, r1]
        sems = [s0, s1]
        c = lax.axis_index("c")
        s = lax.axis_index("s")
        tid = c * NS + s
        zeros16 = jnp.zeros((16,), jnp.float32)

        @pl.loop(0, CH)
        def _zrow(i):
            for j in range(d // 16):
                r0[i, pl.ds(j * 16, 16)] = zeros16

        for k in range(RPS // CH):
            pltpu.sync_copy(r0, acc_sh.at[pl.ds(s * RPS + k * CH, CH)])
        plsc.subcore_barrier()

        @pl.loop(0, NCHUNK // GRP)
        def _grp(g):
            pltpu.sync_copy(src_hbm.at[tid, pl.ds(g * GRP, GRP)], idxs_v)
            pltpu.sync_copy(dst_hbm.at[tid, pl.ds(g * GRP, GRP)], idxd_v)
            descs = [
                pltpu.async_copy(y_hbm.at[idxs_v.at[b]], rows[b], sems[b])
                for b in range(2)
            ]
            for t in range(GRP):
                b = t % 2
                descs[b].wait()
                pltpu.sync_copy(rows[b], acc_sh.at[idxd_v.at[t]], add=True)
                if t + 2 < GRP:
                    descs[b] = pltpu.async_copy(y_hbm.at[idxs_v.at[t + 2]],
                                                rows[b], sems[b])

        plsc.subcore_barrier()
        pltpu.sync_copy(acc_sh.at[pl.ds(s * RPS, RPS)],
                        out_hbm.at[c, pl.ds(s * RPS, RPS)])

    return pl.kernel(
        body,
        out_type=jax.ShapeDtypeStruct((NC, NPAD, d), jnp.float32),
        mesh=_MESH,
        scratch_types=[
            pltpu.VMEM((GRP, CH), jnp.int32),
            pltpu.VMEM((GRP, CH), jnp.int32),
            pltpu.VMEM((CH, d), jnp.float32),
            pltpu.VMEM((CH, d), jnp.float32),
            pltpu.VMEM_SHARED((NPAD, d), jnp.float32),
            pltpu.SemaphoreType.DMA,
            pltpu.SemaphoreType.DMA,
        ],
    )


_sc_scatter_128 = _make_sc_scatter(IN_DIM)


def _prep_body(deg_ref, x_ref, y_ref, dis_ref):
    deg = jnp.sum(deg_ref[...], axis=0) + 1.0
    dis = lax.rsqrt(deg)
    dis_ref[...] = dis
    y_ref[...] = dis * x_ref[...]


def _tc_prep(deg_parts, xp):
    return pl.pallas_call(
        _prep_body,
        grid=(GRID,),
        in_specs=[
            pl.BlockSpec((NC, RB, 1), lambda r: (0, r, 0)),
            pl.BlockSpec((RB, IN_DIM), lambda r: (r, 0)),
        ],
        out_specs=[
            pl.BlockSpec((RB, IN_DIM), lambda r: (r, 0)),
            pl.BlockSpec((RB, 1), lambda r: (r, 0)),
        ],
        out_shape=[
            jax.ShapeDtypeStruct((NPAD, IN_DIM), jnp.float32),
            jax.ShapeDtypeStruct((NPAD, 1), jnp.float32),
        ],
    )(deg_parts, xp)


def _fused_body(acc_ref, y_ref, dis_ref, w1_ref, b1_ref, w2_ref, ym_ref):
    dis = dis_ref[...]
    px = dis * (jnp.sum(acc_ref[...], axis=0) + y_ref[...])
    h = jnp.dot(px.astype(jnp.bfloat16), w1_ref[...].astype(jnp.bfloat16),
                preferred_element_type=jnp.float32)
    h = jnp.maximum(h + b1_ref[...], 0.0)
    m = jnp.dot(h.astype(jnp.bfloat16), w2_ref[...].astype(jnp.bfloat16),
                preferred_element_type=jnp.float32)
    ym_ref[...] = dis * m


def _tc_fused(acc, y, dis, w1p, b1p, w2p):
    return pl.pallas_call(
        _fused_body,
        grid=(GRID,),
        in_specs=[
            pl.BlockSpec((NC, RB, IN_DIM), lambda r: (0, r, 0)),
            pl.BlockSpec((RB, IN_DIM), lambda r: (r, 0)),
            pl.BlockSpec((RB, 1), lambda r: (r, 0)),
            pl.BlockSpec((IN_DIM, HPAD), lambda r: (0, 0)),
            pl.BlockSpec((1, HPAD), lambda r: (0, 0)),
            pl.BlockSpec((HPAD, L_PAD), lambda r: (0, 0)),
        ],
        out_specs=pl.BlockSpec((RB, L_PAD), lambda r: (r, 0)),
        out_shape=jax.ShapeDtypeStruct((NPAD, L_PAD), jnp.float32),
    )(acc, y, dis, w1p, b1p, w2p)


def _final_body(acc_ref, ym_ref, dis_ref, b2_ref, out_ref):
    acc = dis_ref[...] * (jnp.sum(acc_ref[...], axis=0) + ym_ref[...])
    out_ref[...] = jax.nn.sigmoid(acc + b2_ref[...])[:, :L_DIM]


def _tc_final(acc, ym, dis, b2r):
    return pl.pallas_call(
        _final_body,
        grid=(GRID,),
        in_specs=[
            pl.BlockSpec((NC, RB, L_PAD), lambda r: (0, r, 0)),
            pl.BlockSpec((RB, L_PAD), lambda r: (r, 0)),
            pl.BlockSpec((RB, 1), lambda r: (r, 0)),
            pl.BlockSpec((1, L_PAD), lambda r: (0, 0)),
        ],
        out_specs=pl.BlockSpec((RB, L_DIM), lambda r: (r, 0)),
        out_shape=jax.ShapeDtypeStruct((N_NODES, L_DIM), jnp.float32),
    )(acc, ym, dis, b2r)


@jax.jit
def kernel(x, edge_index, W1, b1, W2, b2):
    src = edge_index[0].astype(jnp.int32)
    dst = edge_index[1].astype(jnp.int32)
    pad = N_NODES + (jnp.arange(E_PAD - N_EDGES, dtype=jnp.int32)
                     % (NPAD - N_NODES))
    srcp = jnp.concatenate([src, pad]).reshape(NW, NCHUNK, CH)
    dstp = jnp.concatenate([dst, pad]).reshape(NW, NCHUNK, CH)

    w1p = jnp.pad(W1.astype(jnp.float32), ((0, 0), (0, HPAD - HIDDEN)))
    b1p = jnp.pad(b1.astype(jnp.float32), (0, HPAD - HIDDEN)).reshape(1, HPAD)
    w2p = jnp.pad(W2.astype(jnp.float32),
                  ((0, HPAD - HIDDEN), (0, L_PAD - L_DIM)))
    b2r = jnp.pad(b2.astype(jnp.float32), (0, L_PAD - L_DIM)).reshape(1, L_PAD)

    xp = jnp.pad(x.astype(jnp.float32), ((0, NPAD - N_NODES), (0, 0)))
    deg_parts = _sc_degree(dstp).reshape(NC, NPAD, 1)
    y, dis = _tc_prep(deg_parts, xp)
    acc1 = _sc_scatter_128(y, srcp, dstp)
    ym = _tc_fused(acc1, y, dis, w1p, b1p, w2p)
    acc2 = _sc_scatter_128(ym, srcp, dstp)
    return _tc_final(acc2, ym, dis, b2r)
